# Initial kernel scaffold; baseline (speedup 1.0000x reference)
import jax
import jax.numpy as jnp
from jax.experimental import pallas as pl


def kernel(x, w1, w2):
    raise NotImplementedError("write your pallas kernel here")


# speedup vs baseline: 1.2141x; 1.2141x over previous
import functools

import jax
import jax.numpy as jnp
from jax.experimental import pallas as pl
from jax.experimental.pallas import tpu as pltpu

_MIB = 1024 * 1024


def _se_block_kernel(x_ref, w1t_ref, w2t_ref, o_ref, *, inv_hw):
    x = x_ref[...]

    pooled = jnp.sum(x, axis=-1, dtype=jnp.float32) * inv_hw

    y1 = jnp.dot(pooled, w1t_ref[...].astype(jnp.float32),
                 preferred_element_type=jnp.float32)
    y1 = jnp.clip(y1, 0.0, 6.0)
    y2 = jnp.dot(y1, w2t_ref[...].astype(jnp.float32),
                 preferred_element_type=jnp.float32)
    scale = jax.nn.sigmoid(y2)

    o_ref[...] = x * scale[:, :, None].astype(x.dtype)


def _pick_batch_block(B, slab_bytes):
    for bb in (8, 4, 2, 1):
        if B % bb == 0 and bb * slab_bytes <= 8 * _MIB:
            return bb
    return 1


def kernel(x, w1, w2):
    B, C, H, W = x.shape
    R = w1.shape[0]
    HW = H * W
    itemsize = jnp.dtype(x.dtype).itemsize

    x_flat = x.reshape(B, C, HW)
    w1t = w1.T
    w2t = w2.T

    bb = _pick_batch_block(B, C * HW * itemsize)
    grid = B // bb

    body = functools.partial(_se_block_kernel, inv_hw=1.0 / float(HW))
    blk_bytes = bb * C * HW * itemsize
    w_bytes = (w1.size + w2.size) * 4
    cost = pl.CostEstimate(
        flops=int(2 * B * C * HW + 4 * B * C * R),
        transcendentals=int(B * C),
        bytes_accessed=int(2 * B * C * HW * itemsize + w_bytes),
    )

    out_flat = pl.pallas_call(
        body,
        out_shape=jax.ShapeDtypeStruct((B, C, HW), x.dtype),
        grid=(grid,),
        in_specs=[
            pl.BlockSpec((bb, C, HW), lambda i: (i, 0, 0)),
            pl.BlockSpec((C, R), lambda i: (0, 0)),
            pl.BlockSpec((R, C), lambda i: (0, 0)),
        ],
        out_specs=pl.BlockSpec((bb, C, HW), lambda i: (i, 0, 0)),
        compiler_params=pltpu.CompilerParams(
            dimension_semantics=("parallel",),
            vmem_limit_bytes=int(min(56 * _MIB,
                                     4 * blk_bytes + bb * C * 16 + 2 * w_bytes
                                     + 2 * _MIB)),
        ),
        cost_estimate=cost,
    )(x_flat, w1t, w2t)

    return out_flat.reshape(B, C, H, W)
